# baseline (speedup 1.0000x reference)
import dataclasses
import functools

import jax
import jax.numpy as jnp
from jax import lax
from jax.experimental import pallas as pl
from jax.experimental.pallas import tpu as pltpu
from jax.experimental.pallas import tpu_sc as plsc

N_CORES = 2
N_SUBCORES = 16
N_WORKERS = N_CORES * N_SUBCORES

D = 1024
B = 4 * 8192
V = 8192
N_OWN = V // N_WORKERS
SUB = 32
N_SUB = N_OWN // SUB
CAP = 1280

_CP = pltpu.CompilerParams()
if "needs_layout_passes" in pltpu.CompilerParams.__dataclass_fields__:
    _CP = dataclasses.replace(_CP, needs_layout_passes=False)


def _sc_gather(table, idx):
    mesh = plsc.VectorSubcoreMesh(core_axis_name="c", subcore_axis_name="s")

    @functools.partial(
        pl.kernel,
        mesh=mesh,
        compiler_params=_CP,
        out_type=jax.ShapeDtypeStruct((B, D), jnp.float32),
        scratch_types=[
            pltpu.VMEM((B,), jnp.int32),
            pltpu.VMEM((2, SUB, D), jnp.float32),
            pltpu.VMEM((CAP + 16,), jnp.int32),
            pltpu.SemaphoreType.DMA,
            pltpu.SemaphoreType.DMA,
            pltpu.SemaphoreType.DMA,
        ],
    )
    def k(table_hbm, idx_hbm, out_hbm, idx_all, stage, lst, r0, r1, wsem):
        rsem = (r0, r1)
        wid = lax.axis_index("s") * N_CORES + lax.axis_index("c")
        row_lo = wid * N_OWN
        pltpu.sync_copy(idx_hbm, idx_all)

        iota16 = lax.iota(jnp.int32, 16)

        def cbody(g, cnt):
            v = idx_all[pl.ds(g * 16, 16)]
            m = (v >= row_lo) & (v < row_lo + N_OWN)
            packed = ((g * 16 + iota16) << 13) | v
            pref = plsc.cumsum(jnp.where(m, 1, 0))
            plsc.store_scatter(lst, [cnt + pref - 1], packed, mask=m)
            return jnp.minimum(cnt + pref[15], CAP)

        cnt = lax.fori_loop(0, B // 16, cbody, jnp.int32(0))
        ng = (cnt + 15) // 16

        def rstart(c, b):
            return pltpu.make_async_copy(
                table_hbm.at[pl.ds(row_lo + c * SUB, SUB)],
                stage.at[b],
                rsem[b],
            )

        rstart(0, 0).start()
        rstart(1, 1).start()

        for c in range(N_SUB):
            b = c % 2
            rstart(c, b).wait()
            c_lo = row_lo + c * SUB

            def sbody(g, nis, b=b, c_lo=c_lo):
                e = lst[pl.ds(g * 16, 16)]
                row = e & 0x1FFF
                pos = lax.shift_right_logical(e, 13)
                valid = (g * 16 + iota16) < cnt
                m = (row >= c_lo) & (row < c_lo + SUB) & valid
                mi = jnp.where(m, 1, 0)
                for kk in range(16):

                    @pl.when(mi[kk] == 1)
                    def _():
                        pltpu.make_async_copy(
                            stage.at[b].at[pl.ds(row[kk] - c_lo, 1)],
                            out_hbm.at[pl.ds(pos[kk], 1)],
                            wsem,
                        ).start()

                return nis + plsc.cumsum(mi)[15]

            n_c = lax.fori_loop(0, ng, sbody, jnp.int32(0))

            def dbody(_, x, b=b):
                pltpu.make_async_copy(
                    stage.at[b].at[pl.ds(0, 1)],
                    out_hbm.at[pl.ds(0, 1)],
                    wsem,
                ).wait()
                return x

            lax.fori_loop(0, n_c, dbody, jnp.int32(0))
            if c + 2 < N_SUB:
                rstart(c + 2, b).start()

    return k(table, idx)


def kernel(positions, positional_embeddings):
    idx = positions.reshape(-1).astype(jnp.int32)
    out = _sc_gather(positional_embeddings, idx)
    return out.reshape(positions.shape + (positional_embeddings.shape[1],))


# speedup vs baseline: 1.7386x; 1.7386x over previous
import functools

import jax
import jax.numpy as jnp
from jax import lax
from jax.experimental import pallas as pl
from jax.experimental.pallas import tpu as pltpu
from jax.experimental.pallas import tpu_sc as plsc

N_CORES = 2
N_SUBCORES = 16
N_WORKERS = N_CORES * N_SUBCORES

D = 1024
B = 4 * 8192
B_PER_W = B // N_WORKERS
CHUNK = 16
NBUF = 4
N_CHUNKS = B_PER_W // CHUNK


def _sc_gather(table, idx):
    mesh = plsc.VectorSubcoreMesh(core_axis_name="c", subcore_axis_name="s")

    @functools.partial(
        pl.kernel,
        mesh=mesh,
        out_type=jax.ShapeDtypeStruct((B, D), jnp.float32),
        scratch_types=[
            pltpu.VMEM((B_PER_W,), jnp.int32),
            pltpu.VMEM((NBUF, CHUNK, D), jnp.float32),
        ]
        + [pltpu.SemaphoreType.DMA] * (2 * NBUF),
    )
    def k(table_hbm, idx_hbm, out_hbm, idx_v, rows_v, *sems):
        gsem, wsem = sems[:NBUF], sems[NBUF:]
        wid = lax.axis_index("s") * N_CORES + lax.axis_index("c")
        base = wid * B_PER_W
        pltpu.sync_copy(idx_hbm.at[pl.ds(base, B_PER_W)], idx_v)

        def gcopy(i, s):
            return pltpu.make_async_copy(
                table_hbm.at[idx_v.at[pl.ds(i * CHUNK, CHUNK)]],
                rows_v.at[s],
                gsem[s],
            )

        def wcopy(i, s):
            return pltpu.make_async_copy(
                rows_v.at[s],
                out_hbm.at[pl.ds(base + i * CHUNK, CHUNK)],
                wsem[s],
            )

        for m in range(NBUF - 1):
            gcopy(m, m).start()
        gcopy(0, 0).wait()
        wcopy(0, 0).start()
        gcopy(NBUF - 1, NBUF - 1).start()

        @pl.loop(0, (N_CHUNKS - NBUF) // NBUF)
        def _(blk):
            ibase = 1 + blk * NBUF
            for kk in range(NBUF):
                i = ibase + kk
                s = (1 + kk) % NBUF
                sp = (s - 1) % NBUF
                gcopy(i, s).wait()
                wcopy(i, s).start()
                wcopy(i - 1, sp).wait()
                gcopy(i + NBUF - 1, sp).start()

        for i in range(N_CHUNKS - NBUF + 1, N_CHUNKS):
            s = i % NBUF
            gcopy(i, s).wait()
            wcopy(i, s).start()
        for i in range(N_CHUNKS - NBUF, N_CHUNKS):
            wcopy(i, i % NBUF).wait()

    return k(table, idx)


def kernel(positions, positional_embeddings):
    idx = positions.reshape(-1).astype(jnp.int32)
    out = _sc_gather(positional_embeddings, idx)
    return out.reshape(positions.shape + (positional_embeddings.shape[1],))


# speedup vs baseline: 1.7566x; 1.0103x over previous
import functools

import jax
import jax.numpy as jnp
from jax import lax
from jax.experimental import pallas as pl
from jax.experimental.pallas import tpu as pltpu
from jax.experimental.pallas import tpu_sc as plsc

N_CORES = 2
N_SUBCORES = 16
N_WORKERS = N_CORES * N_SUBCORES

D = 1024
B = 4 * 8192
B_PER_W = B // N_WORKERS
CHUNK = 16
NBUF = 4
N_CHUNKS = B_PER_W // CHUNK


def _sc_gather(table, idx):
    mesh = plsc.VectorSubcoreMesh(core_axis_name="c", subcore_axis_name="s")

    @functools.partial(
        pl.kernel,
        mesh=mesh,
        out_type=jax.ShapeDtypeStruct((B, D), jnp.float32),
        scratch_types=[
            pltpu.VMEM((B_PER_W,), jnp.int32),
            pltpu.VMEM((NBUF, CHUNK, D), jnp.float32),
        ]
        + [pltpu.SemaphoreType.DMA] * (2 * NBUF),
    )
    def k(table_hbm, idx_hbm, out_hbm, idx_v, rows_v, *sems):
        gsem, wsem = sems[:NBUF], sems[NBUF:]
        wid = lax.axis_index("s") * N_CORES + lax.axis_index("c")
        base = wid * B_PER_W
        pltpu.sync_copy(idx_hbm.at[pl.ds(base, B_PER_W)], idx_v)

        def gcopy(i, s):
            return pltpu.make_async_copy(
                table_hbm.at[idx_v.at[pl.ds(i * CHUNK, CHUNK)]],
                rows_v.at[s],
                gsem[s],
            )

        def wcopy(i, s):
            return pltpu.make_async_copy(
                rows_v.at[s],
                out_hbm.at[pl.ds(base + i * CHUNK, CHUNK)],
                wsem[s],
            )

        for m in range(NBUF - 1):
            gcopy(m, m).start()
        gcopy(0, 0).wait()
        wcopy(0, 0).start()
        gcopy(NBUF - 1, NBUF - 1).start()

        @pl.loop(0, (N_CHUNKS - NBUF) // NBUF)
        def _(blk):
            ibase = 1 + blk * NBUF
            for kk in range(NBUF):
                i = ibase + kk
                s = (1 + kk) % NBUF
                sp = (s - 1) % NBUF
                wcopy(i - 1, sp).wait()
                gcopy(i + NBUF - 1, sp).start()
                gcopy(i, s).wait()
                wcopy(i, s).start()

        for i in range(N_CHUNKS - NBUF + 1, N_CHUNKS):
            s = i % NBUF
            gcopy(i, s).wait()
            wcopy(i, s).start()
        for i in range(N_CHUNKS - NBUF, N_CHUNKS):
            wcopy(i, i % NBUF).wait()

    return k(table, idx)


def kernel(positions, positional_embeddings):
    idx = positions.reshape(-1).astype(jnp.int32)
    out = _sc_gather(positional_embeddings, idx)
    return out.reshape(positions.shape + (positional_embeddings.shape[1],))
